# speedup vs baseline: 3.0489x; 3.0489x over previous
import jax
import jax.numpy as jnp
from jax import lax
from jax.experimental import pallas as pl
from jax.experimental.pallas import tpu as pltpu
from jax.experimental.pallas import tpu_sc as plsc

NC, NS = 2, 16
NW = NC * NS
CHUNK = 128
EMB = 128


def _make_gather(steps):
    mesh = plsc.VectorSubcoreMesh(core_axis_name="c", subcore_axis_name="s")
    b_per_w = steps * CHUNK

    def body(idx_hbm, table_hbm, out_hbm, idx_v, rows_v, gsem):
        wid = lax.axis_index("s") * NC + lax.axis_index("c")
        pltpu.sync_copy(idx_hbm.at[wid], idx_v)
        base = wid * b_per_w

        def step(j, carry):
            pltpu.async_copy(table_hbm.at[idx_v.at[j]], rows_v, gsem).wait()
            pltpu.sync_copy(rows_v, out_hbm.at[pl.ds(base + j * CHUNK, CHUNK)])
            return carry

        lax.fori_loop(0, steps, step, 0)

    return pl.kernel(
        body,
        out_type=jax.ShapeDtypeStruct((NW * b_per_w, EMB), jnp.float32),
        mesh=mesh,
        scratch_types=[
            pltpu.VMEM((steps, CHUNK), jnp.int32),
            pltpu.VMEM((CHUNK, EMB), jnp.float32),
            pltpu.SemaphoreType.DMA,
        ],
    )


def kernel(x, table):
    B, H = x.shape
    total = B * H
    steps = total // (NW * CHUNK)
    idx = x.reshape(NW, steps, CHUNK).astype(jnp.int32)
    out = _make_gather(steps)(idx, table)
    return out.reshape(B, H, EMB)


# speedup vs baseline: 3.4597x; 1.1347x over previous
import jax
import jax.numpy as jnp
from jax import lax
from jax.experimental import pallas as pl
from jax.experimental.pallas import tpu as pltpu
from jax.experimental.pallas import tpu_sc as plsc

NC, NS = 2, 16
NW = NC * NS
CHUNK = 128
EMB = 128


def _make_gather(steps):
    mesh = plsc.VectorSubcoreMesh(core_axis_name="c", subcore_axis_name="s")
    b_per_w = steps * CHUNK

    nbuf = 4
    lead = 2

    def body(idx_hbm, table_hbm, out_hbm, idx_v, rows_v, gsem, wsem):
        wid = lax.axis_index("s") * NC + lax.axis_index("c")
        pltpu.sync_copy(idx_hbm.at[wid], idx_v)
        base = wid * b_per_w

        def fire_gather(g, s):
            pltpu.async_copy(table_hbm.at[idx_v.at[g]], rows_v.at[s], gsem)

        def wait_gather():
            pltpu.make_async_copy(
                table_hbm.at[pl.ds(0, CHUNK)], rows_v.at[0], gsem
            ).wait()

        def fire_write(g, s):
            pltpu.async_copy(
                rows_v.at[s], out_hbm.at[pl.ds(base + g * CHUNK, CHUNK)], wsem
            )

        def wait_write():
            pltpu.make_async_copy(
                rows_v.at[0], out_hbm.at[pl.ds(base, CHUNK)], wsem
            ).wait()

        for g in range(lead):
            fire_gather(g, g % nbuf)

        def step(g, carry):
            s = g % nbuf
            wait_gather()
            fire_write(g, s)

            @pl.when(g >= nbuf - lead)
            def _():
                wait_write()

            @pl.when(g + lead < steps)
            def _():
                fire_gather(g + lead, (g + lead) % nbuf)

            return carry

        lax.fori_loop(0, steps, step, 0)
        for _ in range(nbuf - lead):
            wait_write()

    return pl.kernel(
        body,
        out_type=jax.ShapeDtypeStruct((NW * b_per_w, EMB), jnp.float32),
        mesh=mesh,
        scratch_types=[
            pltpu.VMEM((steps, CHUNK), jnp.int32),
            pltpu.VMEM((nbuf, CHUNK, EMB), jnp.float32),
            pltpu.SemaphoreType.DMA,
            pltpu.SemaphoreType.DMA,
        ],
    )


def kernel(x, table):
    B, H = x.shape
    total = B * H
    steps = total // (NW * CHUNK)
    idx = x.reshape(NW, steps, CHUNK).astype(jnp.int32)
    out = _make_gather(steps)(idx, table)
    return out.reshape(B, H, EMB)
